# Initial kernel scaffold; baseline (speedup 1.0000x reference)
import jax
import jax.numpy as jnp
from jax.experimental import pallas as pl


def kernel(x, edge_index, W0, b0, W1, b1):
    raise NotImplementedError("write your pallas kernel here")


# speedup vs baseline: 5.0244x; 5.0244x over previous
import functools

import jax
import jax.numpy as jnp
from jax import lax
from jax.experimental import pallas as pl
from jax.experimental.pallas import tpu as pltpu
from jax.experimental.pallas import tpu_sc as plsc

N = 10000
E = 320000
D = 128

NC = 2
NS = 16
NW = NC * NS
E_PER_TILE = E // NW
K = 80
STEPS = E_PER_TILE // K
ROWS_PER_SUBCORE = N // NS


def _sc_segment_sum(hc, src, dst):
    mesh = plsc.VectorSubcoreMesh(core_axis_name="c", subcore_axis_name="s")

    @functools.partial(
        pl.kernel,
        out_type=jax.ShapeDtypeStruct((NC, N, D), jnp.float32),
        mesh=mesh,
        scratch_types=[
            pltpu.VMEM((K,), jnp.int32),
            pltpu.VMEM((K,), jnp.int32),
            pltpu.VMEM((K, D), jnp.float32),
            pltpu.VMEM_SHARED((N, D), jnp.float32),
        ],
    )
    def seg(hc_hbm, src_hbm, dst_hbm, out_hbm, src_v, dst_v, rows_v, acc_sh):
        cid = lax.axis_index("c")
        sid = lax.axis_index("s")
        wid = sid * NC + cid

        zero16 = jnp.zeros((16,), jnp.float32)

        @pl.loop(0, K)
        def _(i):
            @pl.loop(0, D, step=16)
            def _(j):
                rows_v[i, pl.ds(j, 16)] = zero16

        row0 = sid * ROWS_PER_SUBCORE
        nfull = ROWS_PER_SUBCORE // K
        rem = ROWS_PER_SUBCORE - nfull * K

        @pl.loop(0, nfull)
        def _(i):
            pltpu.sync_copy(rows_v, acc_sh.at[pl.ds(row0 + i * K, K)])

        pltpu.sync_copy(rows_v.at[pl.ds(0, rem)],
                        acc_sh.at[pl.ds(row0 + nfull * K, rem)])

        plsc.subcore_barrier()

        ebase = wid * E_PER_TILE

        @pl.loop(0, STEPS)
        def _(step):
            e0 = ebase + step * K
            pltpu.sync_copy(src_hbm.at[pl.ds(e0, K)], src_v)
            pltpu.sync_copy(dst_hbm.at[pl.ds(e0, K)], dst_v)
            pltpu.sync_copy(hc_hbm.at[src_v], rows_v)
            pltpu.sync_copy(rows_v, acc_sh.at[dst_v], add=True)

        plsc.subcore_barrier()

        wb = (N // NS) // 8 * 8
        pltpu.sync_copy(acc_sh.at[pl.ds(sid * wb, wb)],
                        out_hbm.at[cid, pl.ds(sid * wb, wb)])

        @pl.when(sid == 0)
        def _():
            pltpu.sync_copy(acc_sh.at[pl.ds(NS * wb, N - NS * wb)],
                            out_hbm.at[cid, pl.ds(NS * wb, N - NS * wb)])

    return seg(hc, src, dst)


def _tc_clip(x):
    def body(x_ref, o_ref):
        xb = x_ref[...]
        n2 = jnp.sum(xb * xb, axis=1, keepdims=True)
        scale = 1.0 / jnp.maximum(jnp.sqrt(n2), 1.0)
        o_ref[...] = xb * scale

    return pl.pallas_call(
        body, out_shape=jax.ShapeDtypeStruct((N, D), jnp.float32))(x)


def _tc_layer0(x, hc, s0, s1, W0, b0):
    def body(x_ref, hc_ref, s0_ref, s1_ref, w_ref, b_ref, o_ref):
        agg = hc_ref[...] + s0_ref[...] + s1_ref[...]
        out0 = jnp.dot(agg, w_ref[...],
                       preferred_element_type=jnp.float32,
                       precision=lax.Precision.HIGHEST)
        h = x_ref[...] + jnp.maximum(out0 + b_ref[...], 0.0)
        n2 = jnp.sum(h * h, axis=1, keepdims=True)
        scale = 1.0 / jnp.maximum(jnp.sqrt(n2), 1.0)
        o_ref[...] = h * scale

    return pl.pallas_call(
        body, out_shape=jax.ShapeDtypeStruct((N, D), jnp.float32))(
            x, hc, s0, s1, W0, b0.reshape(1, D))


def _tc_layer1(hc, s0, s1, W1, b1):
    def body(hc_ref, s0_ref, s1_ref, w_ref, b_ref, o_ref):
        agg = hc_ref[...] + s0_ref[...] + s1_ref[...]
        o_ref[...] = jnp.dot(agg, w_ref[...],
                             preferred_element_type=jnp.float32,
                             precision=lax.Precision.HIGHEST) + b_ref[...]

    return pl.pallas_call(
        body, out_shape=jax.ShapeDtypeStruct((N, D), jnp.float32))(
            hc, s0, s1, W1, b1.reshape(1, D))


def kernel(x, edge_index, W0, b0, W1, b1):
    src = edge_index[0].astype(jnp.int32)
    dst = edge_index[1].astype(jnp.int32)

    hc0 = _tc_clip(x)
    p0 = _sc_segment_sum(hc0, src, dst)
    hc1 = _tc_layer0(x, hc0, p0[0], p0[1], W0, b0)
    p1 = _sc_segment_sum(hc1, src, dst)
    return _tc_layer1(hc1, p1[0], p1[1], W1, b1)


# speedup vs baseline: 9.8458x; 1.9596x over previous
import functools

import jax
import jax.numpy as jnp
from jax import lax
from jax.experimental import pallas as pl
from jax.experimental.pallas import tpu as pltpu
from jax.experimental.pallas import tpu_sc as plsc

N = 10000
E = 320000
D = 128

NC = 2
NS = 16
NW = NC * NS
E_PER_TILE = E // NW
K = 80
STEPS = E_PER_TILE // K
ROWS_PER_SUBCORE = N // NS


def _sc_segment_sum(hc, src, dst):
    mesh = plsc.VectorSubcoreMesh(core_axis_name="c", subcore_axis_name="s")

    @functools.partial(
        pl.kernel,
        out_type=jax.ShapeDtypeStruct((NC, N, D), jnp.float32),
        mesh=mesh,
        scratch_types=[
            pltpu.VMEM((4, K), jnp.int32),
            pltpu.VMEM((4, K), jnp.int32),
            pltpu.VMEM((K, D), jnp.float32),
            pltpu.VMEM((K, D), jnp.float32),
            pltpu.VMEM_SHARED((N, D), jnp.float32),
            pltpu.SemaphoreType.DMA,
            pltpu.SemaphoreType.DMA,
            pltpu.SemaphoreType.DMA,
            pltpu.SemaphoreType.DMA,
            pltpu.SemaphoreType.DMA,
            pltpu.SemaphoreType.DMA,
        ],
    )
    def seg(hc_hbm, src_hbm, dst_hbm, out_hbm,
            src_v, dst_v, rows0_v, rows1_v, acc_sh,
            sem0, sem1, si0, si1, si2, si3):
        semi = (si0, si1, si2, si3)
        cid = lax.axis_index("c")
        sid = lax.axis_index("s")
        wid = sid * NC + cid
        ebase = wid * E_PER_TILE

        def load_idx(chunk, slot, sem):
            e0 = ebase + chunk * K
            pltpu.async_copy(src_hbm.at[pl.ds(e0, K)], src_v.at[slot], sem)
            pltpu.async_copy(dst_hbm.at[pl.ds(e0, K)], dst_v.at[slot], sem)

        def wait_idx(chunk, slot, sem):
            e0 = ebase + chunk * K
            pltpu.make_async_copy(src_hbm.at[pl.ds(e0, K)],
                                  src_v.at[slot], sem).wait()
            pltpu.make_async_copy(dst_hbm.at[pl.ds(e0, K)],
                                  dst_v.at[slot], sem).wait()

        def gather(chunk_slot, rows, sem):
            pltpu.async_copy(hc_hbm.at[src_v.at[chunk_slot]], rows, sem)

        def wait_gather(chunk_slot, rows, sem):
            pltpu.make_async_copy(hc_hbm.at[src_v.at[chunk_slot]], rows,
                                  sem).wait()

        load_idx(0, 0, semi[0])
        load_idx(1, 1, semi[1])
        load_idx(2, 2, semi[2])
        load_idx(3, 3, semi[3])

        zero16 = jnp.zeros((16,), jnp.float32)

        @pl.loop(0, K)
        def _(i):
            @pl.loop(0, D, step=16)
            def _(j):
                rows0_v[i, pl.ds(j, 16)] = zero16

        row0 = sid * ROWS_PER_SUBCORE
        nfull = ROWS_PER_SUBCORE // K
        rem = ROWS_PER_SUBCORE - nfull * K

        @pl.loop(0, nfull)
        def _(i):
            pltpu.sync_copy(rows0_v, acc_sh.at[pl.ds(row0 + i * K, K)])

        pltpu.sync_copy(rows0_v.at[pl.ds(0, rem)],
                        acc_sh.at[pl.ds(row0 + nfull * K, rem)])

        wait_idx(0, 0, semi[0])
        gather(0, rows0_v, sem0)
        plsc.subcore_barrier()

        rows = (rows0_v, rows1_v, rows0_v, rows1_v)
        sems = (sem0, sem1, sem0, sem1)

        @pl.loop(0, STEPS // 4)
        def _(jj):
            j0 = jj * 4
            for u in range(4):
                nxt = (u + 1) % 4
                if u < 3:
                    wait_idx(j0 + u + 1, nxt, semi[nxt])
                    gather(nxt, rows[u + 1], sems[u + 1])
                wait_gather(u, rows[u], sems[u])
                pltpu.sync_copy(rows[u], acc_sh.at[dst_v.at[u]], add=True)
                @pl.when(j0 + u + 4 < STEPS)
                def _():
                    load_idx(j0 + u + 4, u, semi[u])

                if u == 3:
                    @pl.when(j0 + 4 < STEPS)
                    def _():
                        wait_idx(j0 + 4, 0, semi[0])
                        gather(0, rows0_v, sem0)

        wait_gather(0, rows0_v, sem0)
        pltpu.sync_copy(rows0_v, acc_sh.at[dst_v.at[0]], add=True)

        plsc.subcore_barrier()

        wb = (N // NS) // 8 * 8
        pltpu.sync_copy(acc_sh.at[pl.ds(sid * wb, wb)],
                        out_hbm.at[cid, pl.ds(sid * wb, wb)])

        @pl.when(sid == 0)
        def _():
            pltpu.sync_copy(acc_sh.at[pl.ds(NS * wb, N - NS * wb)],
                            out_hbm.at[cid, pl.ds(NS * wb, N - NS * wb)])

    return seg(hc, src, dst)


def _tc_clip(x):
    def body(x_ref, o_ref):
        xb = x_ref[...]
        n2 = jnp.sum(xb * xb, axis=1, keepdims=True)
        scale = 1.0 / jnp.maximum(jnp.sqrt(n2), 1.0)
        o_ref[...] = xb * scale

    return pl.pallas_call(
        body, out_shape=jax.ShapeDtypeStruct((N, D), jnp.float32))(x)


def _tc_layer0(x, hc, s0, s1, W0, b0):
    def body(x_ref, hc_ref, s0_ref, s1_ref, w_ref, b_ref, o_ref):
        agg = hc_ref[...] + s0_ref[...] + s1_ref[...]
        out0 = jnp.dot(agg, w_ref[...],
                       preferred_element_type=jnp.float32,
                       precision=lax.Precision.HIGHEST)
        h = x_ref[...] + jnp.maximum(out0 + b_ref[...], 0.0)
        n2 = jnp.sum(h * h, axis=1, keepdims=True)
        scale = 1.0 / jnp.maximum(jnp.sqrt(n2), 1.0)
        o_ref[...] = h * scale

    return pl.pallas_call(
        body, out_shape=jax.ShapeDtypeStruct((N, D), jnp.float32))(
            x, hc, s0, s1, W0, b0.reshape(1, D))


def _tc_layer1(hc, s0, s1, W1, b1):
    def body(hc_ref, s0_ref, s1_ref, w_ref, b_ref, o_ref):
        agg = hc_ref[...] + s0_ref[...] + s1_ref[...]
        o_ref[...] = jnp.dot(agg, w_ref[...],
                             preferred_element_type=jnp.float32,
                             precision=lax.Precision.HIGHEST) + b_ref[...]

    return pl.pallas_call(
        body, out_shape=jax.ShapeDtypeStruct((N, D), jnp.float32))(
            hc, s0, s1, W1, b1.reshape(1, D))


def kernel(x, edge_index, W0, b0, W1, b1):
    src = edge_index[0].astype(jnp.int32)
    dst = edge_index[1].astype(jnp.int32)

    hc0 = _tc_clip(x)
    p0 = _sc_segment_sum(hc0, src, dst)
    hc1 = _tc_layer0(x, hc0, p0[0], p0[1], W0, b0)
    p1 = _sc_segment_sum(hc1, src, dst)
    return _tc_layer1(hc1, p1[0], p1[1], W1, b1)


# speedup vs baseline: 10.8250x; 1.0994x over previous
import functools

import jax
import jax.numpy as jnp
from jax import lax
from jax.experimental import pallas as pl
from jax.experimental.pallas import tpu as pltpu
from jax.experimental.pallas import tpu_sc as plsc

N = 10000
E = 320000
D = 128

NC = 2
NS = 16
NW = NC * NS
E_PER_TILE = E // NW
K = 128
NFULL = E_PER_TILE // K
TAIL = E_PER_TILE - NFULL * K
ROWS_PER_SUBCORE = N // NS


def _sc_segment_sum(hc, ei):
    mesh = plsc.VectorSubcoreMesh(core_axis_name="c", subcore_axis_name="s")

    @functools.partial(
        pl.kernel,
        out_type=jax.ShapeDtypeStruct((NC, N, D), jnp.float32),
        mesh=mesh,
        scratch_types=[
            pltpu.VMEM((4, 2, K), jnp.int32),
            pltpu.VMEM((2, TAIL), jnp.int32),
            pltpu.VMEM((K, D), jnp.float32),
            pltpu.VMEM((K, D), jnp.float32),
            pltpu.VMEM_SHARED((N, D), jnp.float32),
            pltpu.SemaphoreType.DMA,
            pltpu.SemaphoreType.DMA,
            pltpu.SemaphoreType.DMA,
            pltpu.SemaphoreType.DMA,
            pltpu.SemaphoreType.DMA,
            pltpu.SemaphoreType.DMA,
            pltpu.SemaphoreType.DMA,
        ],
    )
    def seg(hc_hbm, ei_hbm, out_hbm,
            idx_v, tidx_v, rows0_v, rows1_v, acc_sh,
            sem0, sem1, semT, si0, si1, si2, si3):
        semi = (si0, si1, si2, si3)
        cid = lax.axis_index("c")
        sid = lax.axis_index("s")
        wid = sid * NC + cid

        def load_idx(chunk, slot, sem):
            pltpu.async_copy(ei_hbm.at[wid, :, pl.ds(chunk * K, K)],
                             idx_v.at[slot], sem)

        def wait_idx(chunk, slot, sem):
            pltpu.make_async_copy(ei_hbm.at[wid, :, pl.ds(chunk * K, K)],
                                  idx_v.at[slot], sem).wait()

        def gather(slot, rows, sem):
            pltpu.async_copy(hc_hbm.at[idx_v.at[slot, 0]], rows, sem)

        def wait_gather(slot, rows, sem):
            pltpu.make_async_copy(hc_hbm.at[idx_v.at[slot, 0]], rows,
                                  sem).wait()

        load_idx(0, 0, semi[0])
        load_idx(1, 1, semi[1])
        load_idx(2, 2, semi[2])
        load_idx(3, 3, semi[3])

        zero16 = jnp.zeros((16,), jnp.float32)

        @pl.loop(0, K)
        def _(i):
            @pl.loop(0, D, step=16)
            def _(j):
                rows0_v[i, pl.ds(j, 16)] = zero16

        row0 = sid * ROWS_PER_SUBCORE
        nz = ROWS_PER_SUBCORE // K
        rz = ROWS_PER_SUBCORE - nz * K

        @pl.loop(0, nz)
        def _(i):
            pltpu.sync_copy(rows0_v, acc_sh.at[pl.ds(row0 + i * K, K)])

        pltpu.sync_copy(rows0_v.at[pl.ds(0, rz)],
                        acc_sh.at[pl.ds(row0 + nz * K, rz)])

        wait_idx(0, 0, semi[0])
        gather(0, rows0_v, sem0)
        plsc.subcore_barrier()

        rows = (rows0_v, rows1_v, rows0_v, rows1_v)
        sems = (sem0, sem1, sem0, sem1)

        @pl.loop(0, NFULL // 4)
        def _(jj):
            j0 = jj * 4
            for u in range(4):
                nxt = (u + 1) % 4
                if u < 3:
                    wait_idx(j0 + u + 1, nxt, semi[nxt])
                    gather(nxt, rows[u + 1], sems[u + 1])
                wait_gather(u, rows[u], sems[u])
                pltpu.sync_copy(rows[u], acc_sh.at[idx_v.at[u, 1]],
                                add=True)
                @pl.when(j0 + u + 4 < NFULL)
                def _():
                    load_idx(j0 + u + 4, u, semi[u])

                if u == 3:
                    @pl.when(j0 + 4 < NFULL)
                    def _():
                        wait_idx(j0 + 4, 0, semi[0])
                        gather(0, rows0_v, sem0)

        wait_idx(NFULL - 1, 1, semi[1])
        gather(1, rows1_v, sem1)
        pltpu.async_copy(ei_hbm.at[wid, :, pl.ds(NFULL * K, TAIL)],
                         tidx_v, semT)
        wait_gather(0, rows0_v, sem0)
        pltpu.sync_copy(rows0_v, acc_sh.at[idx_v.at[0, 1]], add=True)
        pltpu.make_async_copy(ei_hbm.at[wid, :, pl.ds(NFULL * K, TAIL)],
                              tidx_v, semT).wait()
        pltpu.async_copy(hc_hbm.at[tidx_v.at[0]],
                         rows0_v.at[pl.ds(0, TAIL)], sem0)
        wait_gather(1, rows1_v, sem1)
        pltpu.sync_copy(rows1_v, acc_sh.at[idx_v.at[1, 1]], add=True)
        pltpu.make_async_copy(hc_hbm.at[tidx_v.at[0]],
                              rows0_v.at[pl.ds(0, TAIL)], sem0).wait()
        pltpu.sync_copy(rows0_v.at[pl.ds(0, TAIL)],
                        acc_sh.at[tidx_v.at[1]], add=True)

        plsc.subcore_barrier()

        wb = (N // NS) // 8 * 8
        pltpu.sync_copy(acc_sh.at[pl.ds(sid * wb, wb)],
                        out_hbm.at[cid, pl.ds(sid * wb, wb)])

        @pl.when(sid == 0)
        def _():
            pltpu.sync_copy(acc_sh.at[pl.ds(NS * wb, N - NS * wb)],
                            out_hbm.at[cid, pl.ds(NS * wb, N - NS * wb)])

    return seg(hc, ei)


def _tc_clip(x):
    def body(x_ref, o_ref):
        xb = x_ref[...]
        n2 = jnp.sum(xb * xb, axis=1, keepdims=True)
        scale = 1.0 / jnp.maximum(jnp.sqrt(n2), 1.0)
        o_ref[...] = xb * scale

    return pl.pallas_call(
        body, out_shape=jax.ShapeDtypeStruct((N, D), jnp.float32))(x)


def _tc_layer0(x, hc, s0, s1, W0, b0):
    def body(x_ref, hc_ref, s0_ref, s1_ref, w_ref, b_ref, o_ref):
        agg = hc_ref[...] + s0_ref[...] + s1_ref[...]
        out0 = jnp.dot(agg, w_ref[...],
                       preferred_element_type=jnp.float32,
                       precision=lax.Precision.HIGHEST)
        h = x_ref[...] + jnp.maximum(out0 + b_ref[...], 0.0)
        n2 = jnp.sum(h * h, axis=1, keepdims=True)
        scale = 1.0 / jnp.maximum(jnp.sqrt(n2), 1.0)
        o_ref[...] = h * scale

    return pl.pallas_call(
        body, out_shape=jax.ShapeDtypeStruct((N, D), jnp.float32))(
            x, hc, s0, s1, W0, b0.reshape(1, D))


def _tc_layer1(hc, s0, s1, W1, b1):
    def body(hc_ref, s0_ref, s1_ref, w_ref, b_ref, o_ref):
        agg = hc_ref[...] + s0_ref[...] + s1_ref[...]
        o_ref[...] = jnp.dot(agg, w_ref[...],
                             preferred_element_type=jnp.float32,
                             precision=lax.Precision.HIGHEST) + b_ref[...]

    return pl.pallas_call(
        body, out_shape=jax.ShapeDtypeStruct((N, D), jnp.float32))(
            hc, s0, s1, W1, b1.reshape(1, D))


def kernel(x, edge_index, W0, b0, W1, b1):
    ei = (edge_index.astype(jnp.int32)
          .reshape(2, NW, E_PER_TILE).transpose(1, 0, 2))

    hc0 = _tc_clip(x)
    p0 = _sc_segment_sum(hc0, ei)
    hc1 = _tc_layer0(x, hc0, p0[0], p0[1], W0, b0)
    p1 = _sc_segment_sum(hc1, ei)
    return _tc_layer1(hc1, p1[0], p1[1], W1, b1)


# speedup vs baseline: 12.5889x; 1.1629x over previous
import functools

import jax
import jax.numpy as jnp
from jax import lax
from jax.experimental import pallas as pl
from jax.experimental.pallas import tpu as pltpu
from jax.experimental.pallas import tpu_sc as plsc

N = 10000
E = 320000
D = 128

NC = 2
NS = 16
NW = NC * NS
E_PER_TILE = E // NW
K = 128
NFULL = E_PER_TILE // K
TAIL = E_PER_TILE - NFULL * K
ROWS_PER_SUBCORE = N // NS


def _sc_segment_sum(hc, ei):
    mesh = plsc.VectorSubcoreMesh(core_axis_name="c", subcore_axis_name="s")

    @functools.partial(
        pl.kernel,
        out_type=jax.ShapeDtypeStruct((NC, N, D), jnp.float32),
        mesh=mesh,
        scratch_types=[
            pltpu.VMEM((4, 2, K), jnp.int32),
            pltpu.VMEM((2, TAIL), jnp.int32),
            pltpu.VMEM((K, D), jnp.float32),
            pltpu.VMEM((K, D), jnp.float32),
            pltpu.VMEM_SHARED((N, D), jnp.float32),
            pltpu.SemaphoreType.DMA,
            pltpu.SemaphoreType.DMA,
            pltpu.SemaphoreType.DMA,
            pltpu.SemaphoreType.DMA,
            pltpu.SemaphoreType.DMA,
            pltpu.SemaphoreType.DMA,
            pltpu.SemaphoreType.DMA,
            pltpu.SemaphoreType.DMA,
            pltpu.SemaphoreType.DMA,
        ],
    )
    def seg(hc_hbm, ei_hbm, out_hbm,
            idx_v, tidx_v, rows0_v, rows1_v, acc_sh,
            sem0, sem1, semT, semS0, semS1, si0, si1, si2, si3):
        semi = (si0, si1, si2, si3)
        semS = (semS0, semS1)
        cid = lax.axis_index("c")
        sid = lax.axis_index("s")
        wid = sid * NC + cid

        def load_idx(chunk, slot, sem):
            pltpu.async_copy(ei_hbm.at[wid, :, pl.ds(chunk * K, K)],
                             idx_v.at[slot], sem)

        def wait_idx(chunk, slot, sem):
            pltpu.make_async_copy(ei_hbm.at[wid, :, pl.ds(chunk * K, K)],
                                  idx_v.at[slot], sem).wait()

        def gather(slot, rows, sem):
            pltpu.async_copy(hc_hbm.at[idx_v.at[slot, 0]], rows, sem)

        def wait_gather(slot, rows, sem):
            pltpu.make_async_copy(hc_hbm.at[idx_v.at[slot, 0]], rows,
                                  sem).wait()

        def scatter(slot, rows, sem):
            pltpu.async_copy(rows, acc_sh.at[idx_v.at[slot, 1]], sem,
                             add=True)

        def wait_scatter(slot, rows, sem):
            pltpu.make_async_copy(rows, acc_sh.at[idx_v.at[slot, 1]],
                                  sem).wait()

        load_idx(0, 0, semi[0])
        load_idx(1, 1, semi[1])
        load_idx(2, 2, semi[2])

        zero16 = jnp.zeros((16,), jnp.float32)

        @pl.loop(0, K)
        def _(i):
            @pl.loop(0, D, step=16)
            def _(j):
                rows0_v[i, pl.ds(j, 16)] = zero16

        row0 = sid * ROWS_PER_SUBCORE
        nz = ROWS_PER_SUBCORE // K
        rz = ROWS_PER_SUBCORE - nz * K

        @pl.loop(0, nz)
        def _(i):
            pltpu.sync_copy(rows0_v, acc_sh.at[pl.ds(row0 + i * K, K)])

        pltpu.sync_copy(rows0_v.at[pl.ds(0, rz)],
                        acc_sh.at[pl.ds(row0 + nz * K, rz)])

        wait_idx(0, 0, semi[0])
        gather(0, rows0_v, sem0)
        plsc.subcore_barrier()

        rows = (rows0_v, rows1_v)
        sems = (sem0, sem1)

        @pl.loop(0, NFULL // 4)
        def _(jj):
            j0 = jj * 4
            for u in range(4):
                c = j0 + u
                b = u % 2
                nb = 1 - b
                s_next = (u + 1) % 4
                s_prev = (u + 3) % 4
                wait_idx(c + 1, s_next, semi[s_next])
                if u == 0:
                    @pl.when(jj > 0)
                    def _():
                        wait_scatter(s_prev, rows[nb], semS[nb])
                else:
                    wait_scatter(s_prev, rows[nb], semS[nb])
                gather(s_next, rows[nb], sems[nb])
                @pl.when(c + 3 < NFULL)
                def _():
                    load_idx(c + 3, s_prev, semi[s_prev])
                wait_gather(u, rows[b], sems[b])
                scatter(u, rows[b], semS[b])

        wait_idx(NFULL - 1, 1, semi[1])
        wait_scatter(3, rows1_v, semS1)
        gather(1, rows1_v, sem1)
        pltpu.async_copy(ei_hbm.at[wid, :, pl.ds(NFULL * K, TAIL)],
                         tidx_v, semT)
        wait_gather(0, rows0_v, sem0)
        scatter(0, rows0_v, semS0)
        pltpu.make_async_copy(ei_hbm.at[wid, :, pl.ds(NFULL * K, TAIL)],
                              tidx_v, semT).wait()
        wait_scatter(0, rows0_v, semS0)
        pltpu.async_copy(hc_hbm.at[tidx_v.at[0]],
                         rows0_v.at[pl.ds(0, TAIL)], sem0)
        wait_gather(1, rows1_v, sem1)
        scatter(1, rows1_v, semS1)
        pltpu.make_async_copy(hc_hbm.at[tidx_v.at[0]],
                              rows0_v.at[pl.ds(0, TAIL)], sem0).wait()
        pltpu.sync_copy(rows0_v.at[pl.ds(0, TAIL)],
                        acc_sh.at[tidx_v.at[1]], add=True)
        wait_scatter(1, rows1_v, semS1)

        plsc.subcore_barrier()

        wb = (N // NS) // 8 * 8
        pltpu.sync_copy(acc_sh.at[pl.ds(sid * wb, wb)],
                        out_hbm.at[cid, pl.ds(sid * wb, wb)])

        @pl.when(sid == 0)
        def _():
            pltpu.sync_copy(acc_sh.at[pl.ds(NS * wb, N - NS * wb)],
                            out_hbm.at[cid, pl.ds(NS * wb, N - NS * wb)])

    return seg(hc, ei)


def _tc_clip(x):
    def body(x_ref, o_ref):
        xb = x_ref[...]
        n2 = jnp.sum(xb * xb, axis=1, keepdims=True)
        scale = 1.0 / jnp.maximum(jnp.sqrt(n2), 1.0)
        o_ref[...] = xb * scale

    return pl.pallas_call(
        body, out_shape=jax.ShapeDtypeStruct((N, D), jnp.float32))(x)


def _tc_layer0(x, hc, s0, s1, W0, b0):
    def body(x_ref, hc_ref, s0_ref, s1_ref, w_ref, b_ref, o_ref):
        agg = hc_ref[...] + s0_ref[...] + s1_ref[...]
        out0 = jnp.dot(agg, w_ref[...],
                       preferred_element_type=jnp.float32,
                       precision=lax.Precision.HIGHEST)
        h = x_ref[...] + jnp.maximum(out0 + b_ref[...], 0.0)
        n2 = jnp.sum(h * h, axis=1, keepdims=True)
        scale = 1.0 / jnp.maximum(jnp.sqrt(n2), 1.0)
        o_ref[...] = h * scale

    return pl.pallas_call(
        body, out_shape=jax.ShapeDtypeStruct((N, D), jnp.float32))(
            x, hc, s0, s1, W0, b0.reshape(1, D))


def _tc_layer1(hc, s0, s1, W1, b1):
    def body(hc_ref, s0_ref, s1_ref, w_ref, b_ref, o_ref):
        agg = hc_ref[...] + s0_ref[...] + s1_ref[...]
        o_ref[...] = jnp.dot(agg, w_ref[...],
                             preferred_element_type=jnp.float32,
                             precision=lax.Precision.HIGHEST) + b_ref[...]

    return pl.pallas_call(
        body, out_shape=jax.ShapeDtypeStruct((N, D), jnp.float32))(
            hc, s0, s1, W1, b1.reshape(1, D))


def kernel(x, edge_index, W0, b0, W1, b1):
    ei = (edge_index.astype(jnp.int32)
          .reshape(2, NW, E_PER_TILE).transpose(1, 0, 2))

    hc0 = _tc_clip(x)
    p0 = _sc_segment_sum(hc0, ei)
    hc1 = _tc_layer0(x, hc0, p0[0], p0[1], W0, b0)
    p1 = _sc_segment_sum(hc1, ei)
    return _tc_layer1(hc1, p1[0], p1[1], W1, b1)


# speedup vs baseline: 13.0327x; 1.0353x over previous
import functools

import jax
import jax.numpy as jnp
from jax import lax
from jax.experimental import pallas as pl
from jax.experimental.pallas import tpu as pltpu
from jax.experimental.pallas import tpu_sc as plsc

N = 10000
E = 320000
D = 128

NC = 2
NS = 16
NW = NC * NS
E_PER_TILE = E // NW
K = 96
NFULL = E_PER_TILE // K
TAIL = E_PER_TILE - NFULL * K
ROWS_PER_SUBCORE = N // NS


def _sc_segment_sum(hc, eif, eit):
    mesh = plsc.VectorSubcoreMesh(core_axis_name="c", subcore_axis_name="s")

    @functools.partial(
        pl.kernel,
        out_type=jax.ShapeDtypeStruct((NC, N, D), jnp.float32),
        mesh=mesh,
        scratch_types=[
            pltpu.VMEM((6, 2, K), jnp.int32),
            pltpu.VMEM((2, TAIL), jnp.int32),
            pltpu.VMEM((K, D), jnp.float32),
            pltpu.VMEM((K, D), jnp.float32),
            pltpu.VMEM((K, D), jnp.float32),
            pltpu.VMEM_SHARED((N, D), jnp.float32),
        ] + [pltpu.SemaphoreType.DMA] * 13,
    )
    def seg(hc_hbm, eif_hbm, eit_hbm, out_hbm,
            idx_v, tidx_v, rows0_v, rows1_v, rows2_v, acc_sh,
            semG0, semG1, semG2, semS0, semS1, semS2, semT,
            si0, si1, si2, si3, si4, si5):
        semi = (si0, si1, si2, si3, si4, si5)
        semG = (semG0, semG1, semG2)
        semS = (semS0, semS1, semS2)
        rows = (rows0_v, rows1_v, rows2_v)
        cid = lax.axis_index("c")
        sid = lax.axis_index("s")
        wid = sid * NC + cid

        def load_idx(chunk, slot, sem):
            pltpu.async_copy(eif_hbm.at[wid * NFULL + chunk],
                             idx_v.at[slot], sem)

        def wait_idx(chunk, slot, sem):
            pltpu.make_async_copy(eif_hbm.at[wid * NFULL + chunk],
                                  idx_v.at[slot], sem).wait()

        def gather(slot, rows, sem):
            pltpu.async_copy(hc_hbm.at[idx_v.at[slot, 0]], rows, sem)

        def wait_gather(slot, rows, sem):
            pltpu.make_async_copy(hc_hbm.at[idx_v.at[slot, 0]], rows,
                                  sem).wait()

        def scatter(slot, rows, sem):
            pltpu.async_copy(rows, acc_sh.at[idx_v.at[slot, 1]], sem,
                             add=True)

        def wait_scatter(slot, rows, sem):
            pltpu.make_async_copy(rows, acc_sh.at[idx_v.at[slot, 1]],
                                  sem).wait()

        for c0 in range(5):
            load_idx(c0, c0, semi[c0])

        zero16 = jnp.zeros((16,), jnp.float32)

        @pl.loop(0, K)
        def _(i):
            @pl.loop(0, D, step=16)
            def _(j):
                rows0_v[i, pl.ds(j, 16)] = zero16

        row0 = sid * ROWS_PER_SUBCORE
        nz = ROWS_PER_SUBCORE // K
        rz = ROWS_PER_SUBCORE - nz * K

        @pl.loop(0, nz)
        def _(i):
            pltpu.sync_copy(rows0_v, acc_sh.at[pl.ds(row0 + i * K, K)])

        pltpu.sync_copy(rows0_v.at[pl.ds(0, rz)],
                        acc_sh.at[pl.ds(row0 + nz * K, rz)])

        wait_idx(0, 0, semi[0])
        gather(0, rows0_v, semG[0])
        wait_idx(1, 1, semi[1])
        gather(1, rows1_v, semG[1])
        plsc.subcore_barrier()

        @pl.loop(0, NFULL // 6)
        def _(jj):
            j0 = jj * 6
            for u in range(6):
                c = j0 + u
                b = u % 3
                pb = (u + 2) % 3
                s2 = (u + 2) % 6
                sp = (u + 5) % 6
                wait_idx(c + 2, s2, semi[s2])
                wait_gather(u % 6, rows[b], semG[b])
                if u == 0:
                    @pl.when(jj > 0)
                    def _():
                        wait_scatter(sp, rows[pb], semS[pb])
                else:
                    wait_scatter(sp, rows[pb], semS[pb])
                gather(s2, rows[pb], semG[pb])
                @pl.when(c + 5 < NFULL)
                def _():
                    load_idx(c + 5, sp, semi[sp])
                scatter(u % 6, rows[b], semS[b])

        pltpu.async_copy(eit_hbm.at[wid], tidx_v, semT)
        wait_gather(0, rows0_v, semG0)
        wait_scatter(5, rows2_v, semS2)
        scatter(0, rows0_v, semS0)
        wait_gather(1, rows1_v, semG1)
        pltpu.make_async_copy(eit_hbm.at[wid], tidx_v, semT).wait()
        pltpu.async_copy(hc_hbm.at[tidx_v.at[0]],
                         rows2_v.at[pl.ds(0, TAIL)], semG2)
        scatter(1, rows1_v, semS1)
        pltpu.make_async_copy(hc_hbm.at[tidx_v.at[0]],
                              rows2_v.at[pl.ds(0, TAIL)], semG2).wait()
        pltpu.sync_copy(rows2_v.at[pl.ds(0, TAIL)],
                        acc_sh.at[tidx_v.at[1]], add=True)
        wait_scatter(0, rows0_v, semS0)
        wait_scatter(1, rows1_v, semS1)

        plsc.subcore_barrier()

        wb = (N // NS) // 8 * 8
        pltpu.sync_copy(acc_sh.at[pl.ds(sid * wb, wb)],
                        out_hbm.at[cid, pl.ds(sid * wb, wb)])

        @pl.when(sid == 0)
        def _():
            pltpu.sync_copy(acc_sh.at[pl.ds(NS * wb, N - NS * wb)],
                            out_hbm.at[cid, pl.ds(NS * wb, N - NS * wb)])

    return seg(hc, eif, eit)


def _tc_clip(x):
    def body(x_ref, o_ref):
        xb = x_ref[...]
        n2 = jnp.sum(xb * xb, axis=1, keepdims=True)
        scale = 1.0 / jnp.maximum(jnp.sqrt(n2), 1.0)
        o_ref[...] = xb * scale

    return pl.pallas_call(
        body, out_shape=jax.ShapeDtypeStruct((N, D), jnp.float32))(x)


def _tc_layer0(x, hc, s0, s1, W0, b0):
    def body(x_ref, hc_ref, s0_ref, s1_ref, w_ref, b_ref, o_ref):
        agg = hc_ref[...] + s0_ref[...] + s1_ref[...]
        out0 = jnp.dot(agg, w_ref[...],
                       preferred_element_type=jnp.float32,
                       precision=lax.Precision.HIGHEST)
        h = x_ref[...] + jnp.maximum(out0 + b_ref[...], 0.0)
        n2 = jnp.sum(h * h, axis=1, keepdims=True)
        scale = 1.0 / jnp.maximum(jnp.sqrt(n2), 1.0)
        o_ref[...] = h * scale

    return pl.pallas_call(
        body, out_shape=jax.ShapeDtypeStruct((N, D), jnp.float32))(
            x, hc, s0, s1, W0, b0.reshape(1, D))


def _tc_layer1(hc, s0, s1, W1, b1):
    def body(hc_ref, s0_ref, s1_ref, w_ref, b_ref, o_ref):
        agg = hc_ref[...] + s0_ref[...] + s1_ref[...]
        o_ref[...] = jnp.dot(agg, w_ref[...],
                             preferred_element_type=jnp.float32,
                             precision=lax.Precision.HIGHEST) + b_ref[...]

    return pl.pallas_call(
        body, out_shape=jax.ShapeDtypeStruct((N, D), jnp.float32))(
            hc, s0, s1, W1, b1.reshape(1, D))


def kernel(x, edge_index, W0, b0, W1, b1):
    ei = (edge_index.astype(jnp.int32)
          .reshape(2, NW, E_PER_TILE).transpose(1, 0, 2))
    eif = (ei[:, :, :NFULL * K].reshape(NW, 2, NFULL, K)
           .transpose(0, 2, 1, 3).reshape(NW * NFULL, 2, K))
    eit = ei[:, :, NFULL * K:]

    hc0 = _tc_clip(x)
    p0 = _sc_segment_sum(hc0, eif, eit)
    hc1 = _tc_layer0(x, hc0, p0[0], p0[1], W0, b0)
    p1 = _sc_segment_sum(hc1, eif, eit)
    return _tc_layer1(hc1, p1[0], p1[1], W1, b1)


# speedup vs baseline: 13.7195x; 1.0527x over previous
import functools

import jax
import jax.numpy as jnp
from jax import lax
from jax.experimental import pallas as pl
from jax.experimental.pallas import tpu as pltpu
from jax.experimental.pallas import tpu_sc as plsc

N = 10000
E = 320000
D = 128

NC = 2
NS = 16
NW = NC * NS
E_PER_TILE = E // NW
K = 96
NFULL = E_PER_TILE // K
TAIL = E_PER_TILE - NFULL * K
ROWS_PER_SUBCORE = N // NS


def _sc_segment_sum(hc, ef):
    mesh = plsc.VectorSubcoreMesh(core_axis_name="c", subcore_axis_name="s")

    @functools.partial(
        pl.kernel,
        out_type=jax.ShapeDtypeStruct((NC, N, D), jnp.float32),
        mesh=mesh,
        scratch_types=[
            pltpu.VMEM((6, 2, K), jnp.int32),
            pltpu.VMEM((2, TAIL), jnp.int32),
            pltpu.VMEM((K, D), jnp.float32),
            pltpu.VMEM((K, D), jnp.float32),
            pltpu.VMEM((K, D), jnp.float32),
            pltpu.VMEM_SHARED((N, D), jnp.float32),
        ] + [pltpu.SemaphoreType.DMA] * 13,
    )
    def seg(hc_hbm, ef_hbm, out_hbm,
            idx_v, tidx_v, rows0_v, rows1_v, rows2_v, acc_sh,
            semG0, semG1, semG2, semS0, semS1, semS2, semT,
            si0, si1, si2, si3, si4, si5):
        semi = (si0, si1, si2, si3, si4, si5)
        semG = (semG0, semG1, semG2)
        semS = (semS0, semS1, semS2)
        rows = (rows0_v, rows1_v, rows2_v)
        cid = lax.axis_index("c")
        sid = lax.axis_index("s")
        wid = sid * NC + cid
        ebase = wid * E_PER_TILE

        def load_idx(chunk, slot, sem):
            e0 = ebase + chunk * K
            pltpu.async_copy(ef_hbm.at[pl.ds(e0, K)],
                             idx_v.at[slot, 0], sem)
            pltpu.async_copy(ef_hbm.at[pl.ds(E + e0, K)],
                             idx_v.at[slot, 1], sem)

        def wait_idx(chunk, slot, sem):
            e0 = ebase + chunk * K
            pltpu.make_async_copy(ef_hbm.at[pl.ds(e0, K)],
                                  idx_v.at[slot, 0], sem).wait()
            pltpu.make_async_copy(ef_hbm.at[pl.ds(E + e0, K)],
                                  idx_v.at[slot, 1], sem).wait()

        def gather(slot, rows, sem):
            pltpu.async_copy(hc_hbm.at[idx_v.at[slot, 0]], rows, sem)

        def wait_gather(slot, rows, sem):
            pltpu.make_async_copy(hc_hbm.at[idx_v.at[slot, 0]], rows,
                                  sem).wait()

        def scatter(slot, rows, sem):
            pltpu.async_copy(rows, acc_sh.at[idx_v.at[slot, 1]], sem,
                             add=True)

        def wait_scatter(slot, rows, sem):
            pltpu.make_async_copy(rows, acc_sh.at[idx_v.at[slot, 1]],
                                  sem).wait()

        for c0 in range(5):
            load_idx(c0, c0, semi[c0])

        zero16 = jnp.zeros((16,), jnp.float32)

        @pl.loop(0, K)
        def _(i):
            @pl.loop(0, D, step=16)
            def _(j):
                rows0_v[i, pl.ds(j, 16)] = zero16

        row0 = sid * ROWS_PER_SUBCORE
        nz = ROWS_PER_SUBCORE // K
        rz = ROWS_PER_SUBCORE - nz * K

        @pl.loop(0, nz)
        def _(i):
            pltpu.sync_copy(rows0_v, acc_sh.at[pl.ds(row0 + i * K, K)])

        pltpu.sync_copy(rows0_v.at[pl.ds(0, rz)],
                        acc_sh.at[pl.ds(row0 + nz * K, rz)])

        wait_idx(0, 0, semi[0])
        gather(0, rows0_v, semG[0])
        wait_idx(1, 1, semi[1])
        gather(1, rows1_v, semG[1])
        plsc.subcore_barrier()

        @pl.loop(0, NFULL // 6)
        def _(jj):
            j0 = jj * 6
            for u in range(6):
                c = j0 + u
                b = u % 3
                pb = (u + 2) % 3
                s2 = (u + 2) % 6
                sp = (u + 5) % 6
                wait_idx(c + 2, s2, semi[s2])
                wait_gather(u % 6, rows[b], semG[b])
                if u == 0:
                    @pl.when(jj > 0)
                    def _():
                        wait_scatter(sp, rows[pb], semS[pb])
                else:
                    wait_scatter(sp, rows[pb], semS[pb])
                gather(s2, rows[pb], semG[pb])
                @pl.when(c + 5 < NFULL)
                def _():
                    load_idx(c + 5, sp, semi[sp])
                scatter(u % 6, rows[b], semS[b])

        et0 = ebase + NFULL * K
        pltpu.async_copy(ef_hbm.at[pl.ds(et0, TAIL)], tidx_v.at[0], semT)
        pltpu.async_copy(ef_hbm.at[pl.ds(E + et0, TAIL)], tidx_v.at[1],
                         semT)
        wait_gather(0, rows0_v, semG0)
        wait_scatter(5, rows2_v, semS2)
        scatter(0, rows0_v, semS0)
        wait_gather(1, rows1_v, semG1)
        pltpu.make_async_copy(ef_hbm.at[pl.ds(et0, TAIL)], tidx_v.at[0],
                              semT).wait()
        pltpu.make_async_copy(ef_hbm.at[pl.ds(E + et0, TAIL)],
                              tidx_v.at[1], semT).wait()
        pltpu.async_copy(hc_hbm.at[tidx_v.at[0]],
                         rows2_v.at[pl.ds(0, TAIL)], semG2)
        scatter(1, rows1_v, semS1)
        pltpu.make_async_copy(hc_hbm.at[tidx_v.at[0]],
                              rows2_v.at[pl.ds(0, TAIL)], semG2).wait()
        pltpu.sync_copy(rows2_v.at[pl.ds(0, TAIL)],
                        acc_sh.at[tidx_v.at[1]], add=True)
        wait_scatter(0, rows0_v, semS0)
        wait_scatter(1, rows1_v, semS1)

        plsc.subcore_barrier()

        wb = (N // NS) // 8 * 8
        pltpu.sync_copy(acc_sh.at[pl.ds(sid * wb, wb)],
                        out_hbm.at[cid, pl.ds(sid * wb, wb)])

        @pl.when(sid == 0)
        def _():
            pltpu.sync_copy(acc_sh.at[pl.ds(NS * wb, N - NS * wb)],
                            out_hbm.at[cid, pl.ds(NS * wb, N - NS * wb)])

    return seg(hc, ef)


def _tc_clip(x):
    def body(x_ref, o_ref):
        xb = x_ref[...]
        n2 = jnp.sum(xb * xb, axis=1, keepdims=True)
        scale = 1.0 / jnp.maximum(jnp.sqrt(n2), 1.0)
        o_ref[...] = xb * scale

    return pl.pallas_call(
        body, out_shape=jax.ShapeDtypeStruct((N, D), jnp.float32))(x)


def _tc_layer0(x, hc, s0, s1, W0, b0):
    def body(x_ref, hc_ref, s0_ref, s1_ref, w_ref, b_ref, o_ref):
        agg = hc_ref[...] + s0_ref[...] + s1_ref[...]
        out0 = jnp.dot(agg, w_ref[...],
                       preferred_element_type=jnp.float32,
                       precision=lax.Precision.HIGHEST)
        h = x_ref[...] + jnp.maximum(out0 + b_ref[...], 0.0)
        n2 = jnp.sum(h * h, axis=1, keepdims=True)
        scale = 1.0 / jnp.maximum(jnp.sqrt(n2), 1.0)
        o_ref[...] = h * scale

    return pl.pallas_call(
        body, out_shape=jax.ShapeDtypeStruct((N, D), jnp.float32))(
            x, hc, s0, s1, W0, b0.reshape(1, D))


def _tc_layer1(hc, s0, s1, W1, b1):
    def body(hc_ref, s0_ref, s1_ref, w_ref, b_ref, o_ref):
        agg = hc_ref[...] + s0_ref[...] + s1_ref[...]
        o_ref[...] = jnp.dot(agg, w_ref[...],
                             preferred_element_type=jnp.float32,
                             precision=lax.Precision.HIGHEST) + b_ref[...]

    return pl.pallas_call(
        body, out_shape=jax.ShapeDtypeStruct((N, D), jnp.float32))(
            hc, s0, s1, W1, b1.reshape(1, D))


def kernel(x, edge_index, W0, b0, W1, b1):
    ef = edge_index.astype(jnp.int32).reshape(2 * E)

    hc0 = _tc_clip(x)
    p0 = _sc_segment_sum(hc0, ef)
    hc1 = _tc_layer0(x, hc0, p0[0], p0[1], W0, b0)
    p1 = _sc_segment_sum(hc1, ef)
    return _tc_layer1(hc1, p1[0], p1[1], W1, b1)


# speedup vs baseline: 14.0562x; 1.0245x over previous
import functools

import jax
import jax.numpy as jnp
from jax import lax
from jax.experimental import pallas as pl
from jax.experimental.pallas import tpu as pltpu
from jax.experimental.pallas import tpu_sc as plsc

N = 10000
E = 320000
D = 128

NC = 2
NS = 16
NW = NC * NS
E_PER_TILE = E // NW
K = 96
NFULL = E_PER_TILE // K
TAIL = E_PER_TILE - NFULL * K
ROWS_PER_SUBCORE = N // NS


def _sc_segment_sum(hc, ef):
    mesh = plsc.VectorSubcoreMesh(core_axis_name="c", subcore_axis_name="s")

    @functools.partial(
        pl.kernel,
        out_type=jax.ShapeDtypeStruct((NC, N, D), jnp.float32),
        mesh=mesh,
        scratch_types=[
            pltpu.VMEM((6, 2, K), jnp.int32),
            pltpu.VMEM((2, TAIL), jnp.int32),
            pltpu.VMEM((K, D), jnp.float32),
            pltpu.VMEM((K, D), jnp.float32),
            pltpu.VMEM((K, D), jnp.float32),
            pltpu.VMEM_SHARED((N, D), jnp.float32),
        ] + [pltpu.SemaphoreType.DMA] * 13,
    )
    def seg(hc_hbm, ef_hbm, out_hbm,
            idx_v, tidx_v, rows0_v, rows1_v, rows2_v, acc_sh,
            semG0, semG1, semG2, semS0, semS1, semS2, semT,
            si0, si1, si2, si3, si4, si5):
        semi = (si0, si1, si2, si3, si4, si5)
        semG = (semG0, semG1, semG2)
        semS = (semS0, semS1, semS2)
        rows = (rows0_v, rows1_v, rows2_v)
        cid = lax.axis_index("c")
        sid = lax.axis_index("s")
        wid = sid * NC + cid
        ebase = wid * E_PER_TILE

        def load_idx(chunk, slot, sem):
            e0 = ebase + chunk * K
            pltpu.async_copy(ef_hbm.at[pl.ds(e0, K)],
                             idx_v.at[slot, 0], sem)
            pltpu.async_copy(ef_hbm.at[pl.ds(E + e0, K)],
                             idx_v.at[slot, 1], sem)

        def wait_idx(chunk, slot, sem):
            e0 = ebase + chunk * K
            pltpu.make_async_copy(ef_hbm.at[pl.ds(e0, K)],
                                  idx_v.at[slot, 0], sem).wait()
            pltpu.make_async_copy(ef_hbm.at[pl.ds(E + e0, K)],
                                  idx_v.at[slot, 1], sem).wait()

        def gather(slot, rows, sem):
            pltpu.async_copy(hc_hbm.at[idx_v.at[slot, 0]], rows, sem)

        def wait_gather(slot, rows, sem):
            pltpu.make_async_copy(hc_hbm.at[idx_v.at[slot, 0]], rows,
                                  sem).wait()

        def scatter(slot, rows, sem):
            pltpu.async_copy(rows, acc_sh.at[idx_v.at[slot, 1]], sem,
                             add=True)

        def wait_scatter(slot, rows, sem):
            pltpu.make_async_copy(rows, acc_sh.at[idx_v.at[slot, 1]],
                                  sem).wait()

        for c0 in range(5):
            load_idx(c0, c0, semi[c0])

        zero16 = jnp.zeros((16,), jnp.float32)

        @pl.loop(0, K)
        def _(i):
            @pl.loop(0, D, step=16)
            def _(j):
                rows0_v[i, pl.ds(j, 16)] = zero16

        row0 = sid * ROWS_PER_SUBCORE
        nz = ROWS_PER_SUBCORE // K
        rz = ROWS_PER_SUBCORE - nz * K

        @pl.loop(0, nz)
        def _(i):
            pltpu.sync_copy(rows0_v, acc_sh.at[pl.ds(row0 + i * K, K)])

        pltpu.sync_copy(rows0_v.at[pl.ds(0, rz)],
                        acc_sh.at[pl.ds(row0 + nz * K, rz)])

        wait_idx(0, 0, semi[0])
        gather(0, rows0_v, semG[0])
        wait_idx(1, 1, semi[1])
        gather(1, rows1_v, semG[1])
        plsc.subcore_barrier()

        @pl.loop(0, NFULL // 6)
        def _(jj):
            j0 = jj * 6
            for u in range(6):
                c = j0 + u
                b = u % 3
                pb = (u + 2) % 3
                s2 = (u + 2) % 6
                sp = (u + 5) % 6
                wait_idx(c + 2, s2, semi[s2])
                wait_gather(u % 6, rows[b], semG[b])
                if u == 0:
                    @pl.when(jj > 0)
                    def _():
                        wait_scatter(sp, rows[pb], semS[pb])
                else:
                    wait_scatter(sp, rows[pb], semS[pb])
                gather(s2, rows[pb], semG[pb])
                @pl.when(c + 5 < NFULL)
                def _():
                    load_idx(c + 5, sp, semi[sp])
                scatter(u % 6, rows[b], semS[b])

        et0 = ebase + NFULL * K
        pltpu.async_copy(ef_hbm.at[pl.ds(et0, TAIL)], tidx_v.at[0], semT)
        pltpu.async_copy(ef_hbm.at[pl.ds(E + et0, TAIL)], tidx_v.at[1],
                         semT)
        wait_gather(0, rows0_v, semG0)
        wait_scatter(5, rows2_v, semS2)
        scatter(0, rows0_v, semS0)
        wait_gather(1, rows1_v, semG1)
        pltpu.make_async_copy(ef_hbm.at[pl.ds(et0, TAIL)], tidx_v.at[0],
                              semT).wait()
        pltpu.make_async_copy(ef_hbm.at[pl.ds(E + et0, TAIL)],
                              tidx_v.at[1], semT).wait()
        pltpu.async_copy(hc_hbm.at[tidx_v.at[0]],
                         rows2_v.at[pl.ds(0, TAIL)], semG2)
        scatter(1, rows1_v, semS1)
        pltpu.make_async_copy(hc_hbm.at[tidx_v.at[0]],
                              rows2_v.at[pl.ds(0, TAIL)], semG2).wait()
        pltpu.sync_copy(rows2_v.at[pl.ds(0, TAIL)],
                        acc_sh.at[tidx_v.at[1]], add=True)
        wait_scatter(0, rows0_v, semS0)
        wait_scatter(1, rows1_v, semS1)

        plsc.subcore_barrier()

        wb = (N // NS) // 8 * 8
        pltpu.sync_copy(acc_sh.at[pl.ds(sid * wb, wb)],
                        out_hbm.at[cid, pl.ds(sid * wb, wb)])

        @pl.when(sid == 0)
        def _():
            pltpu.sync_copy(acc_sh.at[pl.ds(NS * wb, N - NS * wb)],
                            out_hbm.at[cid, pl.ds(NS * wb, N - NS * wb)])

    return seg(hc, ef)


_TCG = 10
_BN = N // _TCG

_row_spec = pl.BlockSpec((_BN, D), lambda i: (i, 0))
_p_spec = pl.BlockSpec((NC, _BN, D), lambda i: (0, i, 0))
_w_spec = pl.BlockSpec((D, D), lambda i: (0, 0))
_b_spec = pl.BlockSpec((1, D), lambda i: (0, 0))


def _tc_clip(x):
    def body(x_ref, o_ref):
        xb = x_ref[...]
        n2 = jnp.sum(xb * xb, axis=1, keepdims=True)
        scale = 1.0 / jnp.maximum(jnp.sqrt(n2), 1.0)
        o_ref[...] = xb * scale

    return pl.pallas_call(
        body, grid=(_TCG,), in_specs=[_row_spec], out_specs=_row_spec,
        out_shape=jax.ShapeDtypeStruct((N, D), jnp.float32))(x)


def _tc_layer0(x, hc, p, W0, b0):
    def body(x_ref, hc_ref, p_ref, w_ref, b_ref, o_ref):
        agg = hc_ref[...] + p_ref[0] + p_ref[1]
        out0 = jnp.dot(agg, w_ref[...],
                       preferred_element_type=jnp.float32,
                       precision=lax.Precision.HIGHEST)
        h = x_ref[...] + jnp.maximum(out0 + b_ref[...], 0.0)
        n2 = jnp.sum(h * h, axis=1, keepdims=True)
        scale = 1.0 / jnp.maximum(jnp.sqrt(n2), 1.0)
        o_ref[...] = h * scale

    return pl.pallas_call(
        body, grid=(_TCG,),
        in_specs=[_row_spec, _row_spec, _p_spec, _w_spec, _b_spec],
        out_specs=_row_spec,
        out_shape=jax.ShapeDtypeStruct((N, D), jnp.float32))(
            x, hc, p, W0, b0.reshape(1, D))


def _tc_layer1(hc, p, W1, b1):
    def body(hc_ref, p_ref, w_ref, b_ref, o_ref):
        agg = hc_ref[...] + p_ref[0] + p_ref[1]
        o_ref[...] = jnp.dot(agg, w_ref[...],
                             preferred_element_type=jnp.float32,
                             precision=lax.Precision.HIGHEST) + b_ref[...]

    return pl.pallas_call(
        body, grid=(_TCG,),
        in_specs=[_row_spec, _p_spec, _w_spec, _b_spec],
        out_specs=_row_spec,
        out_shape=jax.ShapeDtypeStruct((N, D), jnp.float32))(
            hc, p, W1, b1.reshape(1, D))


def kernel(x, edge_index, W0, b0, W1, b1):
    ef = edge_index.astype(jnp.int32).reshape(2 * E)

    hc0 = _tc_clip(x)
    p0 = _sc_segment_sum(hc0, ef)
    hc1 = _tc_layer0(x, hc0, p0, W0, b0)
    p1 = _sc_segment_sum(hc1, ef)
    return _tc_layer1(hc1, p1, W1, b1)


# speedup vs baseline: 14.5228x; 1.0332x over previous
import functools

import jax
import jax.numpy as jnp
from jax import lax
from jax.experimental import pallas as pl
from jax.experimental.pallas import tpu as pltpu
from jax.experimental.pallas import tpu_sc as plsc

N = 10000
E = 320000
D = 128

NC = 2
NS = 16
NW = NC * NS
E_PER_TILE = E // NW
K = 96
NFULL = E_PER_TILE // K
TAIL = E_PER_TILE - NFULL * K
ROWS_PER_SUBCORE = N // NS


def _sc_segment_sum(hc, ef):
    mesh = plsc.VectorSubcoreMesh(core_axis_name="c", subcore_axis_name="s")

    @functools.partial(
        pl.kernel,
        out_type=jax.ShapeDtypeStruct((NC, N, D), jnp.float32),
        mesh=mesh,
        scratch_types=[
            pltpu.VMEM((6, 2, K), jnp.int32),
            pltpu.VMEM((2, TAIL), jnp.int32),
            pltpu.VMEM((K, D), jnp.float32),
            pltpu.VMEM((K, D), jnp.float32),
            pltpu.VMEM((K, D), jnp.float32),
            pltpu.VMEM_SHARED((N, D), jnp.float32),
        ] + [pltpu.SemaphoreType.DMA] * 13,
    )
    def seg(hc_hbm, ef_hbm, out_hbm,
            idx_v, tidx_v, rows0_v, rows1_v, rows2_v, acc_sh,
            semG0, semG1, semG2, semS0, semS1, semS2, semT,
            si0, si1, si2, si3, si4, si5):
        semi = (si0, si1, si2, si3, si4, si5)
        semG = (semG0, semG1, semG2)
        semS = (semS0, semS1, semS2)
        rows = (rows0_v, rows1_v, rows2_v)
        cid = lax.axis_index("c")
        sid = lax.axis_index("s")
        wid = sid * NC + cid
        ebase = wid * E_PER_TILE

        def load_idx(chunk, slot, sem):
            e0 = ebase + chunk * K
            pltpu.async_copy(ef_hbm.at[pl.ds(e0, K)],
                             idx_v.at[slot, 0], sem)
            pltpu.async_copy(ef_hbm.at[pl.ds(E + e0, K)],
                             idx_v.at[slot, 1], sem)

        def wait_idx(chunk, slot, sem):
            e0 = ebase + chunk * K
            pltpu.make_async_copy(ef_hbm.at[pl.ds(e0, K)],
                                  idx_v.at[slot, 0], sem).wait()
            pltpu.make_async_copy(ef_hbm.at[pl.ds(E + e0, K)],
                                  idx_v.at[slot, 1], sem).wait()

        def gather(slot, rows, sem):
            pltpu.async_copy(hc_hbm.at[idx_v.at[slot, 0]], rows, sem)

        def wait_gather(slot, rows, sem):
            pltpu.make_async_copy(hc_hbm.at[idx_v.at[slot, 0]], rows,
                                  sem).wait()

        def scatter(slot, rows, sem):
            pltpu.async_copy(rows, acc_sh.at[idx_v.at[slot, 1]], sem,
                             add=True)

        def wait_scatter(slot, rows, sem):
            pltpu.make_async_copy(rows, acc_sh.at[idx_v.at[slot, 1]],
                                  sem).wait()

        for c0 in range(5):
            load_idx(c0, c0, semi[c0])

        zero16 = jnp.zeros((16,), jnp.float32)

        @pl.loop(0, K)
        def _(i):
            @pl.loop(0, D, step=16)
            def _(j):
                rows0_v[i, pl.ds(j, 16)] = zero16

        row0 = sid * ROWS_PER_SUBCORE
        nz = ROWS_PER_SUBCORE // K
        rz = ROWS_PER_SUBCORE - nz * K

        @pl.loop(0, nz)
        def _(i):
            pltpu.sync_copy(rows0_v, acc_sh.at[pl.ds(row0 + i * K, K)])

        pltpu.sync_copy(rows0_v.at[pl.ds(0, rz)],
                        acc_sh.at[pl.ds(row0 + nz * K, rz)])

        wait_idx(0, 0, semi[0])
        gather(0, rows0_v, semG[0])
        wait_idx(1, 1, semi[1])
        gather(1, rows1_v, semG[1])
        plsc.subcore_barrier()

        @pl.loop(0, NFULL // 6)
        def _(jj):
            j0 = jj * 6
            for u in range(6):
                c = j0 + u
                b = u % 3
                pb = (u + 2) % 3
                s2 = (u + 2) % 6
                sp = (u + 5) % 6
                wait_idx(c + 2, s2, semi[s2])
                wait_gather(u % 6, rows[b], semG[b])
                if u == 0:
                    @pl.when(jj > 0)
                    def _():
                        wait_scatter(sp, rows[pb], semS[pb])
                else:
                    wait_scatter(sp, rows[pb], semS[pb])
                gather(s2, rows[pb], semG[pb])
                @pl.when(c + 5 < NFULL)
                def _():
                    load_idx(c + 5, sp, semi[sp])
                scatter(u % 6, rows[b], semS[b])

        et0 = ebase + NFULL * K
        pltpu.async_copy(ef_hbm.at[pl.ds(et0, TAIL)], tidx_v.at[0], semT)
        pltpu.async_copy(ef_hbm.at[pl.ds(E + et0, TAIL)], tidx_v.at[1],
                         semT)
        wait_gather(0, rows0_v, semG0)
        wait_scatter(5, rows2_v, semS2)
        scatter(0, rows0_v, semS0)
        wait_gather(1, rows1_v, semG1)
        pltpu.make_async_copy(ef_hbm.at[pl.ds(et0, TAIL)], tidx_v.at[0],
                              semT).wait()
        pltpu.make_async_copy(ef_hbm.at[pl.ds(E + et0, TAIL)],
                              tidx_v.at[1], semT).wait()
        pltpu.async_copy(hc_hbm.at[tidx_v.at[0]],
                         rows2_v.at[pl.ds(0, TAIL)], semG2)
        scatter(1, rows1_v, semS1)
        pltpu.make_async_copy(hc_hbm.at[tidx_v.at[0]],
                              rows2_v.at[pl.ds(0, TAIL)], semG2).wait()
        pltpu.sync_copy(rows2_v.at[pl.ds(0, TAIL)],
                        acc_sh.at[tidx_v.at[1]], add=True)
        wait_scatter(0, rows0_v, semS0)
        wait_scatter(1, rows1_v, semS1)

        plsc.subcore_barrier()

        wb = (N // NS) // 8 * 8
        pltpu.sync_copy(acc_sh.at[pl.ds(sid * wb, wb)],
                        out_hbm.at[cid, pl.ds(sid * wb, wb)])

        @pl.when(sid == 0)
        def _():
            pltpu.sync_copy(acc_sh.at[pl.ds(NS * wb, N - NS * wb)],
                            out_hbm.at[cid, pl.ds(NS * wb, N - NS * wb)])

    return seg(hc, ef)


def _tc_clip(x):
    def body(x_ref, o_ref):
        xb = x_ref[...]
        n2 = jnp.sum(xb * xb, axis=1, keepdims=True)
        scale = 1.0 / jnp.maximum(jnp.sqrt(n2), 1.0)
        o_ref[...] = xb * scale

    return pl.pallas_call(
        body, out_shape=jax.ShapeDtypeStruct((N, D), jnp.float32))(x)


def _tc_layer0(x, hc, p, W0, b0):
    def body(x_ref, hc_ref, p_ref, w_ref, b_ref, o_ref):
        agg = hc_ref[...] + p_ref[0] + p_ref[1]
        out0 = jnp.dot(agg, w_ref[...],
                       preferred_element_type=jnp.float32,
                       precision=lax.Precision.HIGHEST)
        h = x_ref[...] + jnp.maximum(out0 + b_ref[...], 0.0)
        n2 = jnp.sum(h * h, axis=1, keepdims=True)
        scale = 1.0 / jnp.maximum(jnp.sqrt(n2), 1.0)
        o_ref[...] = h * scale

    return pl.pallas_call(
        body, out_shape=jax.ShapeDtypeStruct((N, D), jnp.float32))(
            x, hc, p, W0, b0.reshape(1, D))


def _tc_layer1(hc, p, W1, b1):
    def body(hc_ref, p_ref, w_ref, b_ref, o_ref):
        agg = hc_ref[...] + p_ref[0] + p_ref[1]
        o_ref[...] = jnp.dot(agg, w_ref[...],
                             preferred_element_type=jnp.float32,
                             precision=lax.Precision.HIGHEST) + b_ref[...]

    return pl.pallas_call(
        body, out_shape=jax.ShapeDtypeStruct((N, D), jnp.float32))(
            hc, p, W1, b1.reshape(1, D))


def kernel(x, edge_index, W0, b0, W1, b1):
    ef = edge_index.astype(jnp.int32).reshape(2 * E)

    hc0 = _tc_clip(x)
    p0 = _sc_segment_sum(hc0, ef)
    hc1 = _tc_layer0(x, hc0, p0, W0, b0)
    p1 = _sc_segment_sum(hc1, ef)
    return _tc_layer1(hc1, p1, W1, b1)


# speedup vs baseline: 14.8361x; 1.0216x over previous
import functools

import jax
import jax.numpy as jnp
from jax import lax
from jax.experimental import pallas as pl
from jax.experimental.pallas import tpu as pltpu
from jax.experimental.pallas import tpu_sc as plsc

N = 10000
E = 320000
D = 128

NC = 2
NS = 16
NW = NC * NS
E_PER_TILE = E // NW
K = 96
NFULL = E_PER_TILE // K
TAIL = E_PER_TILE - NFULL * K
ROWS_PER_SUBCORE = N // NS


def _sc_segment_sum(hc, ef):
    mesh = plsc.VectorSubcoreMesh(core_axis_name="c", subcore_axis_name="s")

    @functools.partial(
        pl.kernel,
        out_type=jax.ShapeDtypeStruct((NC, N, D), jnp.float32),
        mesh=mesh,
        scratch_types=[
            pltpu.VMEM((6, 2, K), jnp.int32),
            pltpu.VMEM((2, TAIL), jnp.int32),
            pltpu.VMEM((K, D), jnp.float32),
            pltpu.VMEM((K, D), jnp.float32),
            pltpu.VMEM((K, D), jnp.float32),
            pltpu.VMEM_SHARED((N, D), jnp.float32),
        ] + [pltpu.SemaphoreType.DMA] * 13,
    )
    def seg(hc_hbm, ef_hbm, out_hbm,
            idx_v, tidx_v, rows0_v, rows1_v, rows2_v, acc_sh,
            semG0, semG1, semG2, semS0, semS1, semS2, semT,
            si0, si1, si2, si3, si4, si5):
        semi = (si0, si1, si2, si3, si4, si5)
        semG = (semG0, semG1, semG2)
        semS = (semS0, semS1, semS2)
        rows = (rows0_v, rows1_v, rows2_v)
        cid = lax.axis_index("c")
        sid = lax.axis_index("s")
        wid = sid * NC + cid
        ebase = wid * E_PER_TILE

        def load_idx(chunk, slot, sem):
            e0 = ebase + chunk * K
            pltpu.async_copy(ef_hbm.at[pl.ds(e0, K)],
                             idx_v.at[slot, 0], sem)
            pltpu.async_copy(ef_hbm.at[pl.ds(E + e0, K)],
                             idx_v.at[slot, 1], sem)

        def wait_idx(chunk, slot, sem):
            e0 = ebase + chunk * K
            pltpu.make_async_copy(ef_hbm.at[pl.ds(e0, K)],
                                  idx_v.at[slot, 0], sem).wait()
            pltpu.make_async_copy(ef_hbm.at[pl.ds(E + e0, K)],
                                  idx_v.at[slot, 1], sem).wait()

        def gather(slot, rows, sem):
            pltpu.async_copy(hc_hbm.at[idx_v.at[slot, 0]], rows, sem)

        def wait_gather(slot, rows, sem):
            pltpu.make_async_copy(hc_hbm.at[idx_v.at[slot, 0]], rows,
                                  sem).wait()

        def scatter(slot, rows, sem):
            pltpu.async_copy(rows, acc_sh.at[idx_v.at[slot, 1]], sem,
                             add=True)

        def wait_scatter(slot, rows, sem):
            pltpu.make_async_copy(rows, acc_sh.at[idx_v.at[slot, 1]],
                                  sem).wait()

        for c0 in range(5):
            load_idx(c0, c0, semi[c0])

        zero16 = jnp.zeros((16,), jnp.float32)

        @pl.loop(0, K)
        def _(i):
            @pl.loop(0, D, step=16)
            def _(j):
                rows0_v[i, pl.ds(j, 16)] = zero16

        row0 = sid * ROWS_PER_SUBCORE
        nz = ROWS_PER_SUBCORE // K
        rz = ROWS_PER_SUBCORE - nz * K

        for i in range(ROWS_PER_SUBCORE // K):
            pltpu.async_copy(rows0_v, acc_sh.at[pl.ds(row0 + i * K, K)],
                             semG[i % 3])
        pltpu.async_copy(rows0_v.at[pl.ds(0, rz)],
                         acc_sh.at[pl.ds(row0 + nz * K, rz)], semT)
        for i in range(ROWS_PER_SUBCORE // K):
            pltpu.make_async_copy(
                rows0_v, acc_sh.at[pl.ds(row0 + i * K, K)],
                semG[i % 3]).wait()
        pltpu.make_async_copy(rows0_v.at[pl.ds(0, rz)],
                              acc_sh.at[pl.ds(row0 + nz * K, rz)],
                              semT).wait()

        wait_idx(0, 0, semi[0])
        gather(0, rows0_v, semG[0])
        wait_idx(1, 1, semi[1])
        gather(1, rows1_v, semG[1])
        plsc.subcore_barrier()

        @pl.loop(0, NFULL // 6)
        def _(jj):
            j0 = jj * 6
            for u in range(6):
                c = j0 + u
                b = u % 3
                pb = (u + 2) % 3
                s2 = (u + 2) % 6
                sp = (u + 5) % 6
                wait_idx(c + 2, s2, semi[s2])
                wait_gather(u % 6, rows[b], semG[b])
                if u == 0:
                    @pl.when(jj > 0)
                    def _():
                        wait_scatter(sp, rows[pb], semS[pb])
                else:
                    wait_scatter(sp, rows[pb], semS[pb])
                gather(s2, rows[pb], semG[pb])
                @pl.when(c + 5 < NFULL)
                def _():
                    load_idx(c + 5, sp, semi[sp])
                scatter(u % 6, rows[b], semS[b])

        et0 = ebase + NFULL * K
        pltpu.async_copy(ef_hbm.at[pl.ds(et0, TAIL)], tidx_v.at[0], semT)
        pltpu.async_copy(ef_hbm.at[pl.ds(E + et0, TAIL)], tidx_v.at[1],
                         semT)
        wait_gather(0, rows0_v, semG0)
        wait_scatter(5, rows2_v, semS2)
        scatter(0, rows0_v, semS0)
        wait_gather(1, rows1_v, semG1)
        pltpu.make_async_copy(ef_hbm.at[pl.ds(et0, TAIL)], tidx_v.at[0],
                              semT).wait()
        pltpu.make_async_copy(ef_hbm.at[pl.ds(E + et0, TAIL)],
                              tidx_v.at[1], semT).wait()
        pltpu.async_copy(hc_hbm.at[tidx_v.at[0]],
                         rows2_v.at[pl.ds(0, TAIL)], semG2)
        scatter(1, rows1_v, semS1)
        pltpu.make_async_copy(hc_hbm.at[tidx_v.at[0]],
                              rows2_v.at[pl.ds(0, TAIL)], semG2).wait()
        pltpu.sync_copy(rows2_v.at[pl.ds(0, TAIL)],
                        acc_sh.at[tidx_v.at[1]], add=True)
        wait_scatter(0, rows0_v, semS0)
        wait_scatter(1, rows1_v, semS1)

        plsc.subcore_barrier()

        wb = (N // NS) // 8 * 8
        pltpu.sync_copy(acc_sh.at[pl.ds(sid * wb, wb)],
                        out_hbm.at[cid, pl.ds(sid * wb, wb)])

        @pl.when(sid == 0)
        def _():
            pltpu.sync_copy(acc_sh.at[pl.ds(NS * wb, N - NS * wb)],
                            out_hbm.at[cid, pl.ds(NS * wb, N - NS * wb)])

    return seg(hc, ef)


def _tc_clip(x, ei):
    def body(x_ref, e_ref, o_ref, f_ref):
        xb = x_ref[...]
        n2 = jnp.sum(xb * xb, axis=1, keepdims=True)
        scale = 1.0 / jnp.maximum(jnp.sqrt(n2), 1.0)
        o_ref[...] = xb * scale
        f_ref[pl.ds(0, E)] = e_ref[0, :]
        f_ref[pl.ds(E, E)] = e_ref[1, :]

    return pl.pallas_call(
        body, out_shape=(jax.ShapeDtypeStruct((N, D), jnp.float32),
                         jax.ShapeDtypeStruct((2 * E,), jnp.int32)))(x, ei)


def _tc_layer0(x, hc, p, W0, b0):
    def body(x_ref, hc_ref, p_ref, w_ref, b_ref, o_ref):
        agg = hc_ref[...] + p_ref[0] + p_ref[1]
        out0 = jnp.dot(agg, w_ref[...],
                       preferred_element_type=jnp.float32,
                       precision=lax.Precision.HIGHEST)
        h = x_ref[...] + jnp.maximum(out0 + b_ref[...], 0.0)
        n2 = jnp.sum(h * h, axis=1, keepdims=True)
        scale = 1.0 / jnp.maximum(jnp.sqrt(n2), 1.0)
        o_ref[...] = h * scale

    return pl.pallas_call(
        body, out_shape=jax.ShapeDtypeStruct((N, D), jnp.float32))(
            x, hc, p, W0, b0.reshape(1, D))


def _tc_layer1(hc, p, W1, b1):
    def body(hc_ref, p_ref, w_ref, b_ref, o_ref):
        agg = hc_ref[...] + p_ref[0] + p_ref[1]
        o_ref[...] = jnp.dot(agg, w_ref[...],
                             preferred_element_type=jnp.float32,
                             precision=lax.Precision.HIGHEST) + b_ref[...]

    return pl.pallas_call(
        body, out_shape=jax.ShapeDtypeStruct((N, D), jnp.float32))(
            hc, p, W1, b1.reshape(1, D))


def kernel(x, edge_index, W0, b0, W1, b1):
    hc0, ef = _tc_clip(x, edge_index.astype(jnp.int32))
    p0 = _sc_segment_sum(hc0, ef)
    hc1 = _tc_layer0(x, hc0, p0, W0, b0)
    p1 = _sc_segment_sum(hc1, ef)
    return _tc_layer1(hc1, p1, W1, b1)
